# baseline (speedup 1.0000x reference)
import functools

import jax
import jax.numpy as jnp
from jax import lax
from jax.experimental import pallas as pl
from jax.experimental.pallas import tpu as pltpu
from jax.experimental.pallas import tpu_sc as plsc

NC = 2
NS = 16
NW = NC * NS

CHUNK = 128


@functools.partial(jax.jit, static_argnames=("b_per_w", "d"))
def _sc_gather(idx, table, b_per_w, d):
    num_chunks = b_per_w // CHUNK
    b_total = idx.shape[0]

    mesh = plsc.VectorSubcoreMesh(core_axis_name="c", subcore_axis_name="s")

    @functools.partial(
        pl.kernel,
        out_type=jax.ShapeDtypeStruct((b_total, d), jnp.float32),
        mesh=mesh,
        scratch_types=[
            pltpu.VMEM((CHUNK,), jnp.int32),
            pltpu.VMEM((CHUNK, d), jnp.float32),
            pltpu.SemaphoreType.DMA,
        ],
        compiler_params=pltpu.CompilerParams(use_tc_tiling_on_sc=False),
    )
    def k(idx_hbm, table_hbm, out_hbm, idx_v, rows_v, sem):
        wid = lax.axis_index("s") * NC + lax.axis_index("c")
        base = wid * b_per_w

        def body(j, carry):
            off = base + j * CHUNK
            pltpu.sync_copy(idx_hbm.at[pl.ds(off, CHUNK)], idx_v)
            pltpu.async_copy(table_hbm.at[idx_v], rows_v, sem).wait()
            pltpu.sync_copy(rows_v, out_hbm.at[pl.ds(off, CHUNK)])
            return carry

        lax.fori_loop(0, num_chunks, body, 0)

    return k(idx, table)


def kernel(x, table):
    b_total = x.size
    d = table.shape[1]
    idx = x.reshape(b_total).astype(jnp.int32)
    assert b_total % NW == 0
    b_per_w = b_total // NW
    assert b_per_w % CHUNK == 0
    out = _sc_gather(idx, table, b_per_w, d)
    return out.reshape(x.shape + (d,))


# speedup vs baseline: 1.1960x; 1.1960x over previous
import functools

import jax
import jax.numpy as jnp
from jax import lax
from jax.experimental import pallas as pl
from jax.experimental.pallas import tpu as pltpu
from jax.experimental.pallas import tpu_sc as plsc

NC = 2
NS = 16
NW = NC * NS

CHUNK = 512
NB = 2


@functools.partial(jax.jit, static_argnames=("b_per_w", "d"))
def _sc_gather(idx, table, b_per_w, d):
    num_chunks = b_per_w // CHUNK
    nsteps = num_chunks // NB
    b_total = idx.shape[0]

    mesh = plsc.VectorSubcoreMesh(core_axis_name="c", subcore_axis_name="s")

    @functools.partial(
        pl.kernel,
        out_type=jax.ShapeDtypeStruct((b_total, d), jnp.float32),
        mesh=mesh,
        scratch_types=[
            pltpu.VMEM((b_per_w,), jnp.int32),
            pltpu.VMEM((NB, CHUNK, d), jnp.float32),
            pltpu.SemaphoreType.DMA((NB,)),
            pltpu.SemaphoreType.DMA((NB,)),
        ],
        compiler_params=pltpu.CompilerParams(use_tc_tiling_on_sc=False),
    )
    def k(idx_hbm, table_hbm, out_hbm, idx_v, rows_v, gsem, wsem):
        wid = lax.axis_index("s") * NC + lax.axis_index("c")
        base = wid * b_per_w

        pltpu.sync_copy(idx_hbm.at[pl.ds(base, b_per_w)], idx_v)

        def gather(j, b):
            return pltpu.make_async_copy(
                table_hbm.at[idx_v.at[pl.ds(j * CHUNK, CHUNK)]],
                rows_v.at[b],
                gsem.at[b],
            )

        def writeback(j, b):
            return pltpu.make_async_copy(
                rows_v.at[b],
                out_hbm.at[pl.ds(base + j * CHUNK, CHUNK)],
                wsem.at[b],
            )

        for b in range(NB):
            gather(b, b).start()

        def body(g, carry):
            for b in range(NB):
                j = g * NB + b
                gather(j, b).wait()
                writeback(j, b).start()
                writeback(j, b).wait()

                @pl.when(g < nsteps - 1)
                def _():
                    gather(j + NB, b).start()

            return carry

        lax.fori_loop(0, nsteps, body, 0)

    return k(idx, table)


def kernel(x, table):
    b_total = x.size
    d = table.shape[1]
    idx = x.reshape(b_total).astype(jnp.int32)
    assert b_total % NW == 0
    b_per_w = b_total // NW
    assert b_per_w % (CHUNK * NB) == 0
    out = _sc_gather(idx, table, b_per_w, d)
    return out.reshape(x.shape + (d,))
